# Initial kernel scaffold; baseline (speedup 1.0000x reference)
import jax
import jax.numpy as jnp
from jax.experimental import pallas as pl


def kernel(src, lengths, table):
    raise NotImplementedError("write your pallas kernel here")


# speedup vs baseline: 11.6977x; 11.6977x over previous
import jax
import jax.numpy as jnp
from jax import lax
from jax.experimental import pallas as pl
from jax.experimental.pallas import tpu as pltpu
from jax.experimental.pallas import tpu_sc as plsc

L = 200
B = 4096
D = 64
NC = 2
NS = 16
NW = NC * NS
BPW = B // NW
NV = D // 16


def _body(src_hbm, len_hbm, table_hbm, out_hbm,
          idx_v, gath_v, acc_v, len_v, sem0, sem1):
    wid = lax.axis_index("s") * NC + lax.axis_index("c")
    base = wid * BPW

    pltpu.sync_copy(src_hbm.at[:, pl.ds(base, BPW)], idx_v)
    pltpu.sync_copy(len_hbm.at[pl.ds(base, BPW)], len_v)

    zeros = jnp.zeros((16,), jnp.float32)

    @plsc.parallel_loop(0, BPW, 1, unroll=8)
    def _zero(e):
        for r in range(NV):
            acc_v[e, pl.ds(r * 16, 16)] = zeros

    def fire(l, buf, sem):
        pltpu.async_copy(table_hbm.at[idx_v.at[l]], gath_v.at[buf], sem)

    def wait(l, buf, sem):
        pltpu.make_async_copy(table_hbm.at[idx_v.at[l]], gath_v.at[buf],
                              sem).wait()

    def accumulate(buf):
        @plsc.parallel_loop(0, BPW, 1, unroll=8)
        def _acc(e):
            for r in range(NV):
                v = gath_v[buf, e, pl.ds(r * 16, 16)]
                plsc.addupdate(acc_v.at[e, pl.ds(r * 16, 16)], v)

    fire(0, 0, sem0)
    fire(1, 1, sem1)

    @pl.loop(0, L // 2)
    def _step(t):
        l = t * 2
        wait(l, 0, sem0)
        accumulate(0)

        @pl.when(l + 2 < L)
        def _():
            fire(l + 2, 0, sem0)

        wait(l + 1, 1, sem1)
        accumulate(1)

        @pl.when(l + 3 < L)
        def _():
            fire(l + 3, 1, sem1)

    @plsc.parallel_loop(0, BPW // 16, 1)
    def _scale(eb):
        lv = len_v[pl.ds(eb * 16, 16)].astype(jnp.float32)
        inv = jnp.full((16,), 1.0, jnp.float32) / lv
        for j in range(16):
            vinv = jnp.full((16,), inv[j], jnp.float32)
            e = eb * 16 + j
            for r in range(NV):
                sl = pl.ds(r * 16, 16)
                acc_v[e, sl] = acc_v[e, sl] * vinv

    pltpu.sync_copy(acc_v, out_hbm.at[pl.ds(base, BPW)])


@jax.jit
def kernel(src, lengths, table):
    mesh = plsc.VectorSubcoreMesh(core_axis_name="c", subcore_axis_name="s",
                                  num_cores=NC, num_subcores=NS)
    f = pl.kernel(
        _body,
        out_type=jax.ShapeDtypeStruct((B, D), jnp.float32),
        mesh=mesh,
        scratch_types=[
            pltpu.VMEM((L, BPW), jnp.int32),
            pltpu.VMEM((2, BPW, D), jnp.float32),
            pltpu.VMEM((BPW, D), jnp.float32),
            pltpu.VMEM((BPW,), jnp.int32),
            pltpu.SemaphoreType.DMA,
            pltpu.SemaphoreType.DMA,
        ],
        compiler_params=pltpu.CompilerParams(use_tc_tiling_on_sc=False),
    )
    return f(src.astype(jnp.int32), lengths.astype(jnp.int32), table)


# speedup vs baseline: 13.7885x; 1.1787x over previous
import jax
import jax.numpy as jnp
from jax import lax
from jax.experimental import pallas as pl
from jax.experimental.pallas import tpu as pltpu
from jax.experimental.pallas import tpu_sc as plsc

L = 200
B = 4096
D = 64
NC = 2
NS = 16
NW = NC * NS
BPW = B // NW
NV = D // 16
NBUF = 4


def _body(src_hbm, len_hbm, table_hbm, out_hbm,
          idx_v, gath_v, acc_v, len_v, sem0, sem1, sem2, sem3):
    wid = lax.axis_index("s") * NC + lax.axis_index("c")
    base = wid * BPW

    pltpu.sync_copy(src_hbm.at[:, pl.ds(base, BPW)], idx_v)
    pltpu.sync_copy(len_hbm.at[pl.ds(base, BPW)], len_v)

    zeros = jnp.zeros((16,), jnp.float32)

    @plsc.parallel_loop(0, BPW, 1, unroll=8)
    def _zero(e):
        for r in range(NV):
            acc_v[e, pl.ds(r * 16, 16)] = zeros

    def fire(l, buf, sem):
        pltpu.async_copy(table_hbm.at[idx_v.at[l]], gath_v.at[buf], sem)

    def wait(l, buf, sem):
        pltpu.make_async_copy(table_hbm.at[idx_v.at[l]], gath_v.at[buf],
                              sem).wait()

    def accumulate(buf):
        @plsc.parallel_loop(0, BPW, 1, unroll=8)
        def _acc(e):
            for r in range(NV):
                v = gath_v[buf, e, pl.ds(r * 16, 16)]
                plsc.addupdate(acc_v.at[e, pl.ds(r * 16, 16)], v)

    sems = (sem0, sem1, sem2, sem3)
    for b in range(NBUF):
        fire(b, b, sems[b])

    @pl.loop(0, L // NBUF)
    def _step(t):
        l = t * NBUF
        for b in range(NBUF):
            wait(l + b, b, sems[b])
            accumulate(b)

            @pl.when(l + b + NBUF < L)
            def _():
                fire(l + b + NBUF, b, sems[b])

    @plsc.parallel_loop(0, BPW // 16, 1)
    def _scale(eb):
        lv = len_v[pl.ds(eb * 16, 16)].astype(jnp.float32)
        inv = jnp.full((16,), 1.0, jnp.float32) / lv
        for j in range(16):
            vinv = jnp.full((16,), inv[j], jnp.float32)
            e = eb * 16 + j
            for r in range(NV):
                sl = pl.ds(r * 16, 16)
                acc_v[e, sl] = acc_v[e, sl] * vinv

    pltpu.sync_copy(acc_v, out_hbm.at[pl.ds(base, BPW)])


@jax.jit
def kernel(src, lengths, table):
    mesh = plsc.VectorSubcoreMesh(core_axis_name="c", subcore_axis_name="s",
                                  num_cores=NC, num_subcores=NS)
    f = pl.kernel(
        _body,
        out_type=jax.ShapeDtypeStruct((B, D), jnp.float32),
        mesh=mesh,
        scratch_types=[
            pltpu.VMEM((L, BPW), jnp.int32),
            pltpu.VMEM((NBUF, BPW, D), jnp.float32),
            pltpu.VMEM((BPW, D), jnp.float32),
            pltpu.VMEM((BPW,), jnp.int32),
            pltpu.SemaphoreType.DMA,
            pltpu.SemaphoreType.DMA,
            pltpu.SemaphoreType.DMA,
            pltpu.SemaphoreType.DMA,
        ],
        compiler_params=pltpu.CompilerParams(use_tc_tiling_on_sc=False),
    )
    return f(src.astype(jnp.int32), lengths.astype(jnp.int32), table)


# speedup vs baseline: 16.4106x; 1.1902x over previous
import jax
import jax.numpy as jnp
from jax import lax
from jax.experimental import pallas as pl
from jax.experimental.pallas import tpu as pltpu
from jax.experimental.pallas import tpu_sc as plsc

L = 200
B = 4096
D = 64
NC = 2
NS = 16
NW = NC * NS
BPW = B // NW
NV = D // 16
NBUF = 4


def _body(src_hbm, len_hbm, table_hbm, out_hbm,
          idx_v, gath_v, acc_v, len_v, sem0, sem1, sem2, sem3):
    wid = lax.axis_index("s") * NC + lax.axis_index("c")
    base = wid * BPW

    pltpu.sync_copy(src_hbm.at[:, pl.ds(base, BPW)], idx_v)
    pltpu.sync_copy(len_hbm.at[pl.ds(base, BPW)], len_v)

    zeros = jnp.zeros((16,), jnp.float32)

    @plsc.parallel_loop(0, BPW, 1, unroll=8)
    def _zero(e):
        for r in range(NV):
            acc_v[e, pl.ds(r * 16, 16)] = zeros

    sems = (sem0, sem1, sem2, sem3)

    def fire(l, sem):
        pltpu.async_copy(table_hbm.at[idx_v.at[l]], acc_v, sem, add=True)

    def wait(l, sem):
        pltpu.make_async_copy(table_hbm.at[idx_v.at[l]], acc_v, sem).wait()

    for b in range(NBUF):
        fire(b, sems[b])

    @pl.loop(0, L // NBUF)
    def _step(t):
        l = t * NBUF
        for b in range(NBUF):
            wait(l + b, sems[b])

            @pl.when(l + b + NBUF < L)
            def _():
                fire(l + b + NBUF, sems[b])

    @plsc.parallel_loop(0, BPW // 16, 1)
    def _scale(eb):
        lv = len_v[pl.ds(eb * 16, 16)].astype(jnp.float32)
        inv = jnp.full((16,), 1.0, jnp.float32) / lv
        for j in range(16):
            vinv = jnp.full((16,), inv[j], jnp.float32)
            e = eb * 16 + j
            for r in range(NV):
                sl = pl.ds(r * 16, 16)
                acc_v[e, sl] = acc_v[e, sl] * vinv

    pltpu.sync_copy(acc_v, out_hbm.at[pl.ds(base, BPW)])


@jax.jit
def kernel(src, lengths, table):
    mesh = plsc.VectorSubcoreMesh(core_axis_name="c", subcore_axis_name="s",
                                  num_cores=NC, num_subcores=NS)
    f = pl.kernel(
        _body,
        out_type=jax.ShapeDtypeStruct((B, D), jnp.float32),
        mesh=mesh,
        scratch_types=[
            pltpu.VMEM((L, BPW), jnp.int32),
            pltpu.VMEM((NBUF, BPW, D), jnp.float32),
            pltpu.VMEM((BPW, D), jnp.float32),
            pltpu.VMEM((BPW,), jnp.int32),
            pltpu.SemaphoreType.DMA,
            pltpu.SemaphoreType.DMA,
            pltpu.SemaphoreType.DMA,
            pltpu.SemaphoreType.DMA,
        ],
        compiler_params=pltpu.CompilerParams(use_tc_tiling_on_sc=False),
    )
    return f(src.astype(jnp.int32), lengths.astype(jnp.int32), table)


# speedup vs baseline: 17.9130x; 1.0915x over previous
import jax
import jax.numpy as jnp
from jax import lax
from jax.experimental import pallas as pl
from jax.experimental.pallas import tpu as pltpu
from jax.experimental.pallas import tpu_sc as plsc

L = 200
B = 4096
D = 64
NC = 2
NS = 16
NW = NC * NS
BPW = B // NW
NV = D // 16
NBUF = 8


def _body(src_hbm, len_hbm, table_hbm, out_hbm,
          idx_v, acc_v, len_v, sem0, sem1, sem2, sem3, sem4, sem5, sem6,
          sem7):
    wid = lax.axis_index("s") * NC + lax.axis_index("c")
    base = wid * BPW

    pltpu.sync_copy(src_hbm.at[:, pl.ds(base, BPW)], idx_v)
    pltpu.sync_copy(len_hbm.at[pl.ds(base, BPW)], len_v)

    zeros = jnp.zeros((16,), jnp.float32)

    @plsc.parallel_loop(0, BPW, 1, unroll=8)
    def _zero(e):
        for r in range(NV):
            acc_v[e, pl.ds(r * 16, 16)] = zeros

    sems = (sem0, sem1, sem2, sem3, sem4, sem5, sem6, sem7)

    def fire(l, sem):
        pltpu.async_copy(table_hbm.at[idx_v.at[l]], acc_v, sem, add=True)

    def wait(l, sem):
        pltpu.make_async_copy(table_hbm.at[idx_v.at[l]], acc_v, sem).wait()

    for b in range(NBUF):
        fire(b, sems[b])

    @pl.loop(0, L // NBUF)
    def _step(t):
        l = t * NBUF
        for b in range(NBUF):
            wait(l + b, sems[b])

            @pl.when(l + b + NBUF < L)
            def _():
                fire(l + b + NBUF, sems[b])

    @plsc.parallel_loop(0, BPW // 16, 1)
    def _scale(eb):
        lv = len_v[pl.ds(eb * 16, 16)].astype(jnp.float32)
        inv = jnp.full((16,), 1.0, jnp.float32) / lv
        for j in range(16):
            vinv = jnp.full((16,), inv[j], jnp.float32)
            e = eb * 16 + j
            for r in range(NV):
                sl = pl.ds(r * 16, 16)
                acc_v[e, sl] = acc_v[e, sl] * vinv

    pltpu.sync_copy(acc_v, out_hbm.at[pl.ds(base, BPW)])


@jax.jit
def kernel(src, lengths, table):
    mesh = plsc.VectorSubcoreMesh(core_axis_name="c", subcore_axis_name="s",
                                  num_cores=NC, num_subcores=NS)
    f = pl.kernel(
        _body,
        out_type=jax.ShapeDtypeStruct((B, D), jnp.float32),
        mesh=mesh,
        scratch_types=[
            pltpu.VMEM((L, BPW), jnp.int32),
            pltpu.VMEM((BPW, D), jnp.float32),
            pltpu.VMEM((BPW,), jnp.int32),
        ] + [pltpu.SemaphoreType.DMA] * NBUF,
        compiler_params=pltpu.CompilerParams(use_tc_tiling_on_sc=False),
    )
    return f(src.astype(jnp.int32), lengths.astype(jnp.int32), table)
